# baseline (speedup 1.0000x reference)
import functools

import jax
import jax.numpy as jnp
from jax import lax
from jax.experimental import pallas as pl
from jax.experimental.pallas import tpu as pltpu
from jax.experimental.pallas import tpu_sc as plsc

N = 10000
E = 320000
D = 128
H = 16

NC = 2
NS = 16
NW = NC * NS
CHUNK = 128
GDEPTH = 4
NBUF = 2 * GDEPTH
EPW = 80
EDGES_PAD = NW * EPW * CHUNK
ROWS_PER_TILE = 632
ACC_ROWS = NS * ROWS_PER_TILE

_mesh = plsc.VectorSubcoreMesh(core_axis_name="c", subcore_axis_name="s")
_sc_params = pltpu.CompilerParams(use_tc_tiling_on_sc=False)


@functools.partial(
    pl.kernel,
    out_type=jax.ShapeDtypeStruct((NC, N), jnp.float32),
    mesh=_mesh,
    compiler_params=_sc_params,
    scratch_types=[
        pltpu.VMEM((EPW, CHUNK), jnp.int32),
        pltpu.VMEM((CHUNK,), jnp.float32),
        pltpu.VMEM_SHARED((ACC_ROWS,), jnp.float32),
        pltpu.SemaphoreType.DMA,
    ],
)
def _deg_kernel(dst_hbm, ones_hbm, zeros_hbm, out_hbm, dst_v, ones_v, deg_sh,
                sem):
    cid = lax.axis_index("c")
    sid = lax.axis_index("s")
    wid = sid * NC + cid
    rows = ROWS_PER_TILE
    pltpu.sync_copy(zeros_hbm.at[pl.ds(sid * rows, rows)],
                    deg_sh.at[pl.ds(sid * rows, rows)])
    pltpu.sync_copy(dst_hbm.at[wid], dst_v)
    pltpu.sync_copy(ones_hbm, ones_v)
    plsc.subcore_barrier()

    def fire(j, carry):
        pltpu.async_copy(ones_v, deg_sh.at[dst_v.at[j]], sem, add=True)
        return carry

    lax.fori_loop(0, EPW, fire, 0)

    def drain(j, carry):
        pltpu.make_async_copy(ones_v, deg_sh.at[dst_v.at[j]], sem).wait()
        return carry

    lax.fori_loop(0, EPW, drain, 0)
    plsc.subcore_barrier()

    @pl.when(sid == 0)
    def _():
        pltpu.sync_copy(deg_sh.at[pl.ds(0, N)], out_hbm.at[cid])


def _enc_body(x_ref, w_ref, degp_ref, y_ref, dinv_ref):
    deg = degp_ref[0] + degp_ref[1] + 1.0
    dinv = lax.rsqrt(deg)
    xw = jnp.dot(x_ref[...], w_ref[...], preferred_element_type=jnp.float32)
    y_ref[...] = xw * dinv
    dinv_ref[...] = dinv


_enc_call = pl.pallas_call(
    _enc_body,
    out_shape=[
        jax.ShapeDtypeStruct((N, H), jnp.float32),
        jax.ShapeDtypeStruct((N, 1), jnp.float32),
    ],
)


@functools.partial(
    pl.kernel,
    out_type=jax.ShapeDtypeStruct((NC, ACC_ROWS, H), jnp.float32),
    mesh=_mesh,
    compiler_params=_sc_params,
    scratch_types=[
        pltpu.VMEM((EPW, CHUNK), jnp.int32),
        pltpu.VMEM((EPW, CHUNK), jnp.int32),
        pltpu.VMEM((NBUF, CHUNK, H), jnp.float32),
        pltpu.VMEM_SHARED((ACC_ROWS, H), jnp.float32),
        [pltpu.SemaphoreType.DMA] * NBUF,
        [pltpu.SemaphoreType.DMA] * NBUF,
    ],
)
def _msg_kernel(y_hbm, src_hbm, dst_hbm, zeros_hbm, out_hbm,
                src_v, dst_v, buf, acc_sh, gs, ss):
    cid = lax.axis_index("c")
    sid = lax.axis_index("s")
    wid = sid * NC + (1 - cid)
    rows = ROWS_PER_TILE
    pltpu.sync_copy(zeros_hbm.at[pl.ds(sid * rows, rows)],
                    acc_sh.at[pl.ds(sid * rows, rows)])
    pltpu.sync_copy(src_hbm.at[wid], src_v)
    pltpu.sync_copy(dst_hbm.at[wid], dst_v)
    plsc.subcore_barrier()

    def gather(j, b):
        pltpu.async_copy(y_hbm.at[src_v.at[j]], buf.at[b], gs[b])

    def gwait(j, b):
        pltpu.make_async_copy(y_hbm.at[src_v.at[j]], buf.at[b], gs[b]).wait()

    def scat(j, b):
        pltpu.async_copy(buf.at[b], acc_sh.at[dst_v.at[j]], ss[b], add=True)

    def swait(j, b):
        pltpu.make_async_copy(buf.at[b], acc_sh.at[dst_v.at[j]], ss[b]).wait()

    for b in range(GDEPTH):
        gather(b, b)
    for b in range(GDEPTH):
        gwait(b, b)
        scat(b, b)
        gather(b + GDEPTH, b + GDEPTH)
    for b in range(GDEPTH, NBUF):
        gwait(b, b)
        scat(b, b)
        swait(b - GDEPTH, b - GDEPTH)
        gather(b + GDEPTH, b - GDEPTH)

    def body(io, carry):
        for b in range(NBUF):
            j = io * NBUF + b
            bn = (b + GDEPTH) % NBUF
            gwait(j, b)
            scat(j, b)
            swait(j - GDEPTH, bn)
            gather(j + GDEPTH, bn)
        return carry

    lax.fori_loop(1, EPW // NBUF - 1, body, 0)
    base = EPW - NBUF
    for b in range(NBUF):
        gwait(base + b, b)
        scat(base + b, b)
        if b < GDEPTH:
            swait(base + b - GDEPTH, b + GDEPTH)
            gather(base + b + GDEPTH, b + GDEPTH)
    for b in range(NBUF):
        swait(base + b, b)
    plsc.subcore_barrier()
    pltpu.sync_copy(acc_sh.at[pl.ds(sid * rows, rows)],
                    out_hbm.at[cid, pl.ds(sid * rows, rows)])


def _dec_body(accp_ref, y_ref, dinv_ref, benc_ref, wdec_ref, bdec_ref,
              xhat_ref, z_ref):
    acc = accp_ref[0, :N, :] + accp_ref[1, :N, :] + y_ref[...]
    t = acc * dinv_ref[...] + benc_ref[...]
    z = jnp.maximum(t, 0.0)
    z_ref[...] = z
    xhat_ref[...] = (
        jnp.dot(z, wdec_ref[...], preferred_element_type=jnp.float32)
        + bdec_ref[...]
    )


_dec_call = pl.pallas_call(
    _dec_body,
    out_shape=[
        jax.ShapeDtypeStruct((N, D), jnp.float32),
        jax.ShapeDtypeStruct((N, H), jnp.float32),
    ],
)


def kernel(x, edge_index, W_enc, b_enc, W_dec, b_dec):
    src = edge_index[0]
    dst = edge_index[1]
    pad = EDGES_PAD - E
    src_p = jnp.concatenate(
        [src, jnp.zeros((pad,), jnp.int32)]).reshape(NW, EPW, CHUNK)
    pad_dst = N + jnp.arange(pad, dtype=jnp.int32) % (ACC_ROWS - N)
    dst_p = jnp.concatenate([dst, pad_dst]).reshape(NW, EPW, CHUNK)
    ones_h = jnp.ones((CHUNK,), jnp.float32)
    zeros1 = jnp.zeros((ACC_ROWS,), jnp.float32)
    zerosh = jnp.zeros((ACC_ROWS, H), jnp.float32)

    degp = _deg_kernel(dst_p, ones_h, zeros1)
    y, dinv = _enc_call(x, W_enc, degp.reshape(NC, N, 1))
    accp = _msg_kernel(y, src_p, dst_p, zerosh)
    x_hat, z = _dec_call(accp, y, dinv,
                         b_enc.reshape(1, H), W_dec, b_dec.reshape(1, D))
    return (x_hat, z)


# speedup vs baseline: 1.2781x; 1.2781x over previous
import functools

import jax
import jax.numpy as jnp
from jax import lax
from jax.experimental import pallas as pl
from jax.experimental.pallas import tpu as pltpu
from jax.experimental.pallas import tpu_sc as plsc

N = 10000
E = 320000
D = 128
H = 16

NC = 2
NS = 16
NW = NC * NS
CHUNK = 128
GDEPTH = 4
NBUF = 2 * GDEPTH
EPW = 80
EDGES_PAD = NW * EPW * CHUNK
ROWS_PER_TILE = 632
ACC_ROWS = NS * ROWS_PER_TILE

_mesh = plsc.VectorSubcoreMesh(core_axis_name="c", subcore_axis_name="s")
_sc_params = pltpu.CompilerParams(use_tc_tiling_on_sc=False)


@functools.partial(
    pl.kernel,
    out_type=jax.ShapeDtypeStruct((NC, N), jnp.float32),
    mesh=_mesh,
    compiler_params=_sc_params,
    scratch_types=[
        pltpu.VMEM((EPW, CHUNK), jnp.int32),
        pltpu.VMEM((CHUNK,), jnp.float32),
        pltpu.VMEM_SHARED((ACC_ROWS,), jnp.float32),
        pltpu.SemaphoreType.DMA,
    ],
)
def _deg_kernel(dst_hbm, ones_hbm, zeros_hbm, out_hbm, dst_v, ones_v, deg_sh,
                sem):
    cid = lax.axis_index("c")
    sid = lax.axis_index("s")
    wid = sid * NC + cid
    rows = ROWS_PER_TILE
    pltpu.sync_copy(zeros_hbm.at[pl.ds(sid * rows, rows)],
                    deg_sh.at[pl.ds(sid * rows, rows)])
    pltpu.sync_copy(dst_hbm.at[wid], dst_v)
    pltpu.sync_copy(ones_hbm, ones_v)
    plsc.subcore_barrier()

    def fire(j, carry):
        pltpu.async_copy(ones_v, deg_sh.at[dst_v.at[j]], sem, add=True)
        return carry

    lax.fori_loop(0, EPW, fire, 0)

    def drain(j, carry):
        pltpu.make_async_copy(ones_v, deg_sh.at[dst_v.at[j]], sem).wait()
        return carry

    lax.fori_loop(0, EPW, drain, 0)
    plsc.subcore_barrier()

    @pl.when(sid == 0)
    def _():
        pltpu.sync_copy(deg_sh.at[pl.ds(0, N)], out_hbm.at[cid])


def _enc_body(x_ref, w_ref, degp_ref, y_ref, dinv_ref):
    deg = degp_ref[0] + degp_ref[1] + 1.0
    dinv = lax.rsqrt(deg)
    xw = jnp.dot(x_ref[...], w_ref[...], preferred_element_type=jnp.float32)
    y_ref[...] = xw * dinv
    dinv_ref[...] = dinv


_enc_call = pl.pallas_call(
    _enc_body,
    out_shape=[
        jax.ShapeDtypeStruct((N, H), jnp.float32),
        jax.ShapeDtypeStruct((N, 1), jnp.float32),
    ],
)


@functools.partial(
    pl.kernel,
    out_type=jax.ShapeDtypeStruct((NC, ACC_ROWS, H), jnp.float32),
    mesh=_mesh,
    compiler_params=_sc_params,
    scratch_types=[
        pltpu.VMEM((EPW, CHUNK), jnp.int32),
        pltpu.VMEM((EPW, CHUNK), jnp.int32),
        pltpu.VMEM((NBUF, CHUNK, H), jnp.float32),
        pltpu.VMEM_SHARED((ACC_ROWS, H), jnp.float32),
        [pltpu.SemaphoreType.DMA] * NBUF,
        [pltpu.SemaphoreType.DMA] * NBUF,
    ],
)
def _msg_kernel(y_hbm, src_hbm, dst_hbm, zeros_hbm, out_hbm,
                src_v, dst_v, buf, acc_sh, gs, ss):
    cid = lax.axis_index("c")
    sid = lax.axis_index("s")
    wid = sid * NC + cid
    rows = ROWS_PER_TILE
    pltpu.sync_copy(zeros_hbm.at[pl.ds(sid * rows, rows)],
                    acc_sh.at[pl.ds(sid * rows, rows)])
    pltpu.sync_copy(src_hbm.at[wid], src_v)
    pltpu.sync_copy(dst_hbm.at[wid], dst_v)
    plsc.subcore_barrier()

    def gather(j, b):
        pltpu.async_copy(y_hbm.at[src_v.at[j]], buf.at[b], gs[b])

    def gwait(j, b):
        pltpu.make_async_copy(y_hbm.at[src_v.at[j]], buf.at[b], gs[b]).wait()

    def scat(j, b):
        pltpu.async_copy(buf.at[b], acc_sh.at[dst_v.at[j]], ss[b], add=True)

    def swait(j, b):
        pltpu.make_async_copy(buf.at[b], acc_sh.at[dst_v.at[j]], ss[b]).wait()

    for b in range(GDEPTH):
        gather(b, b)
    for b in range(GDEPTH):
        gwait(b, b)
        scat(b, b)
        gather(b + GDEPTH, b + GDEPTH)
    for b in range(GDEPTH, NBUF):
        gwait(b, b)
        scat(b, b)
        swait(b - GDEPTH, b - GDEPTH)
        gather(b + GDEPTH, b - GDEPTH)

    def body(io, carry):
        for b in range(NBUF):
            j = io * NBUF + b
            bn = (b + GDEPTH) % NBUF
            gwait(j, b)
            scat(j, b)
            swait(j - GDEPTH, bn)
            gather(j + GDEPTH, bn)
        return carry

    lax.fori_loop(1, EPW // NBUF - 1, body, 0)
    base = EPW - NBUF
    for b in range(NBUF):
        gwait(base + b, b)
        scat(base + b, b)
        if b < GDEPTH:
            swait(base + b - GDEPTH, b + GDEPTH)
            gather(base + b + GDEPTH, b + GDEPTH)
    for b in range(NBUF):
        swait(base + b, b)
    plsc.subcore_barrier()
    pltpu.sync_copy(acc_sh.at[pl.ds(sid * rows, rows)],
                    out_hbm.at[cid, pl.ds(sid * rows, rows)])


def _dec_body(accp_ref, y_ref, dinv_ref, benc_ref, wdec_ref, bdec_ref,
              xhat_ref, z_ref):
    acc = accp_ref[0, :N, :] + accp_ref[1, :N, :] + y_ref[...]
    t = acc * dinv_ref[...] + benc_ref[...]
    z = jnp.maximum(t, 0.0)
    z_ref[...] = z
    xhat_ref[...] = (
        jnp.dot(z, wdec_ref[...], preferred_element_type=jnp.float32)
        + bdec_ref[...]
    )


_dec_call = pl.pallas_call(
    _dec_body,
    out_shape=[
        jax.ShapeDtypeStruct((N, D), jnp.float32),
        jax.ShapeDtypeStruct((N, H), jnp.float32),
    ],
)


def kernel(x, edge_index, W_enc, b_enc, W_dec, b_dec):
    src = edge_index[0]
    dst = edge_index[1]
    pad = EDGES_PAD - E
    iota = jnp.arange(pad, dtype=jnp.int32).reshape(NW, pad // NW)
    src_p = jnp.concatenate(
        [src.reshape(NW, E // NW), iota % N], axis=1).reshape(NW, EPW, CHUNK)
    dst_p = jnp.concatenate(
        [dst.reshape(NW, E // NW), N + iota % (ACC_ROWS - N)],
        axis=1).reshape(NW, EPW, CHUNK)
    ones_h = jnp.ones((CHUNK,), jnp.float32)
    zeros1 = jnp.zeros((ACC_ROWS,), jnp.float32)
    zerosh = jnp.zeros((ACC_ROWS, H), jnp.float32)

    degp = _deg_kernel(dst_p, ones_h, zeros1)
    y, dinv = _enc_call(x, W_enc, degp.reshape(NC, N, 1))
    accp = _msg_kernel(y, src_p, dst_p, zerosh)
    x_hat, z = _dec_call(accp, y, dinv,
                         b_enc.reshape(1, H), W_dec, b_dec.reshape(1, D))
    return (x_hat, z)


# speedup vs baseline: 1.4252x; 1.1151x over previous
import functools

import jax
import jax.numpy as jnp
from jax import lax
from jax.experimental import pallas as pl
from jax.experimental.pallas import tpu as pltpu
from jax.experimental.pallas import tpu_sc as plsc

N = 10000
E = 320000
D = 128
H = 16

NC = 2
NS = 16
NW = NC * NS
CHUNK = 128
GDEPTH = 4
NBUF = 2 * GDEPTH
EPN = E // NW
NF = EPN // CHUNK
TAIL = EPN - NF * CHUNK
ROWS_PER_TILE = 632
ACC_ROWS = NS * ROWS_PER_TILE

_mesh = plsc.VectorSubcoreMesh(core_axis_name="c", subcore_axis_name="s")
_sc_params = pltpu.CompilerParams(use_tc_tiling_on_sc=False)


@functools.partial(
    pl.kernel,
    out_type=jax.ShapeDtypeStruct((NC, N), jnp.float32),
    mesh=_mesh,
    compiler_params=_sc_params,
    scratch_types=[
        pltpu.VMEM((EPN,), jnp.int32),
        pltpu.VMEM((CHUNK,), jnp.float32),
        pltpu.VMEM_SHARED((ACC_ROWS,), jnp.float32),
        pltpu.SemaphoreType.DMA,
    ],
)
def _deg_kernel(edge_hbm, ones_hbm, zeros_hbm, out_hbm, dst_v, ones_v, deg_sh,
                sem):
    cid = lax.axis_index("c")
    sid = lax.axis_index("s")
    wid = sid * NC + cid
    rows = ROWS_PER_TILE
    pltpu.sync_copy(zeros_hbm.at[pl.ds(sid * rows, rows)],
                    deg_sh.at[pl.ds(sid * rows, rows)])
    pltpu.sync_copy(edge_hbm.at[1, pl.ds(wid * EPN, EPN)], dst_v)
    pltpu.sync_copy(ones_hbm, ones_v)
    plsc.subcore_barrier()

    def didx(j):
        return deg_sh.at[dst_v.at[pl.ds(j * CHUNK, CHUNK)]]

    def fire(j, carry):
        pltpu.async_copy(ones_v, didx(j), sem, add=True)
        return carry

    lax.fori_loop(0, NF, fire, 0)
    tail_dst = deg_sh.at[dst_v.at[pl.ds(NF * CHUNK, TAIL)]]
    pltpu.async_copy(ones_v.at[pl.ds(0, TAIL)], tail_dst, sem, add=True)

    def drain(j, carry):
        pltpu.make_async_copy(ones_v, didx(j), sem).wait()
        return carry

    lax.fori_loop(0, NF, drain, 0)
    pltpu.make_async_copy(ones_v.at[pl.ds(0, TAIL)], tail_dst, sem).wait()
    plsc.subcore_barrier()

    @pl.when(sid == 0)
    def _():
        pltpu.sync_copy(deg_sh.at[pl.ds(0, N)], out_hbm.at[cid])


def _enc_body(x_ref, w_ref, degp_ref, y_ref, dinv_ref):
    deg = degp_ref[0] + degp_ref[1] + 1.0
    dinv = lax.rsqrt(deg)
    xw = jnp.dot(x_ref[...], w_ref[...], preferred_element_type=jnp.float32)
    y_ref[...] = xw * dinv
    dinv_ref[...] = dinv


_enc_call = pl.pallas_call(
    _enc_body,
    out_shape=[
        jax.ShapeDtypeStruct((N, H), jnp.float32),
        jax.ShapeDtypeStruct((N, 1), jnp.float32),
    ],
)


@functools.partial(
    pl.kernel,
    out_type=jax.ShapeDtypeStruct((NC, ACC_ROWS, H), jnp.float32),
    mesh=_mesh,
    compiler_params=_sc_params,
    scratch_types=[
        pltpu.VMEM((EPN,), jnp.int32),
        pltpu.VMEM((EPN,), jnp.int32),
        pltpu.VMEM((NBUF, CHUNK, H), jnp.float32),
        pltpu.VMEM((TAIL, H), jnp.float32),
        pltpu.VMEM_SHARED((ACC_ROWS, H), jnp.float32),
        [pltpu.SemaphoreType.DMA] * NBUF,
        [pltpu.SemaphoreType.DMA] * NBUF,
    ],
)
def _msg_kernel(y_hbm, edge_hbm, zeros_hbm, out_hbm,
                src_v, dst_v, buf, tbuf, acc_sh, gs, ss):
    cid = lax.axis_index("c")
    sid = lax.axis_index("s")
    wid = sid * NC + cid
    rows = ROWS_PER_TILE
    pltpu.sync_copy(zeros_hbm.at[pl.ds(sid * rows, rows)],
                    acc_sh.at[pl.ds(sid * rows, rows)])
    pltpu.sync_copy(edge_hbm.at[0, pl.ds(wid * EPN, EPN)], src_v)
    pltpu.sync_copy(edge_hbm.at[1, pl.ds(wid * EPN, EPN)], dst_v)
    plsc.subcore_barrier()

    def gather(j, b):
        pltpu.async_copy(y_hbm.at[src_v.at[pl.ds(j * CHUNK, CHUNK)]],
                         buf.at[b], gs[b])

    def gwait(j, b):
        pltpu.make_async_copy(y_hbm.at[src_v.at[pl.ds(j * CHUNK, CHUNK)]],
                              buf.at[b], gs[b]).wait()

    def scat(j, b):
        pltpu.async_copy(buf.at[b], acc_sh.at[dst_v.at[pl.ds(j * CHUNK, CHUNK)]],
                         ss[b], add=True)

    def swait(j, b):
        pltpu.make_async_copy(buf.at[b],
                              acc_sh.at[dst_v.at[pl.ds(j * CHUNK, CHUNK)]],
                              ss[b]).wait()

    for b in range(GDEPTH):
        gather(b, b)
    for b in range(GDEPTH):
        gwait(b, b)
        scat(b, b)
        gather(b + GDEPTH, b + GDEPTH)
    for b in range(GDEPTH, NBUF):
        gwait(b, b)
        scat(b, b)
        swait(b - GDEPTH, b - GDEPTH)
        gather(b + GDEPTH, b - GDEPTH)

    def body(io, carry):
        for b in range(NBUF):
            j = io * NBUF + b
            bn = (b + GDEPTH) % NBUF
            gwait(j, b)
            scat(j, b)
            swait(j - GDEPTH, bn)
            gather(j + GDEPTH, bn)
        return carry

    lax.fori_loop(1, NF // NBUF, body, 0)
    for j in range(NF - NF % NBUF, NF):
        b = j % NBUF
        gwait(j, b)
        scat(j, b)
        if j + GDEPTH < NF:
            bn = (j + GDEPTH) % NBUF
            swait(j - GDEPTH, bn)
            gather(j + GDEPTH, bn)
    for k in range(NBUF):
        j = NF - NBUF + k
        swait(j, j % NBUF)
    pltpu.sync_copy(y_hbm.at[src_v.at[pl.ds(NF * CHUNK, TAIL)]], tbuf)
    pltpu.sync_copy(tbuf, acc_sh.at[dst_v.at[pl.ds(NF * CHUNK, TAIL)]],
                    add=True)
    plsc.subcore_barrier()
    pltpu.sync_copy(acc_sh.at[pl.ds(sid * rows, rows)],
                    out_hbm.at[cid, pl.ds(sid * rows, rows)])


def _dec_body(accp_ref, y_ref, dinv_ref, benc_ref, wdec_ref, bdec_ref,
              xhat_ref, z_ref):
    acc = accp_ref[0, :N, :] + accp_ref[1, :N, :] + y_ref[...]
    t = acc * dinv_ref[...] + benc_ref[...]
    z = jnp.maximum(t, 0.0)
    z_ref[...] = z
    xhat_ref[...] = (
        jnp.dot(z, wdec_ref[...], preferred_element_type=jnp.float32)
        + bdec_ref[...]
    )


_dec_call = pl.pallas_call(
    _dec_body,
    out_shape=[
        jax.ShapeDtypeStruct((N, D), jnp.float32),
        jax.ShapeDtypeStruct((N, H), jnp.float32),
    ],
)


def kernel(x, edge_index, W_enc, b_enc, W_dec, b_dec):
    ones_h = jnp.ones((CHUNK,), jnp.float32)
    zeros1 = jnp.zeros((ACC_ROWS,), jnp.float32)
    zerosh = jnp.zeros((ACC_ROWS, H), jnp.float32)

    degp = _deg_kernel(edge_index, ones_h, zeros1)
    y, dinv = _enc_call(x, W_enc, degp.reshape(NC, N, 1))
    accp = _msg_kernel(y, edge_index, zerosh)
    x_hat, z = _dec_call(accp, y, dinv,
                         b_enc.reshape(1, H), W_dec, b_dec.reshape(1, D))
    return (x_hat, z)


# speedup vs baseline: 1.6717x; 1.1729x over previous
import functools

import jax
import jax.numpy as jnp
from jax import lax
from jax.experimental import pallas as pl
from jax.experimental.pallas import tpu as pltpu
from jax.experimental.pallas import tpu_sc as plsc

N = 10000
E = 320000
D = 128
H = 16

NC = 2
NS = 16
NW = NC * NS
CHUNK = 128
GDEPTH = 4
NBUF = 2 * GDEPTH
EPN = E // NW
NF = EPN // CHUNK
TAIL = EPN - NF * CHUNK
ROWS_PER_TILE = 632
ACC_ROWS = NS * ROWS_PER_TILE

_mesh = plsc.VectorSubcoreMesh(core_axis_name="c", subcore_axis_name="s")
_sc_params = pltpu.CompilerParams(use_tc_tiling_on_sc=False)


@functools.partial(
    pl.kernel,
    out_type=jax.ShapeDtypeStruct((NC, N), jnp.float32),
    mesh=_mesh,
    compiler_params=_sc_params,
    scratch_types=[
        pltpu.VMEM((EPN,), jnp.int32),
        pltpu.VMEM((CHUNK,), jnp.float32),
        pltpu.VMEM_SHARED((ACC_ROWS,), jnp.float32),
        pltpu.SemaphoreType.DMA,
    ],
)
def _deg_kernel(edge_hbm, ones_hbm, zeros_hbm, out_hbm, dst_v, ones_v, deg_sh,
                sem):
    cid = lax.axis_index("c")
    sid = lax.axis_index("s")
    wid = sid * NC + cid
    rows = ROWS_PER_TILE
    pltpu.sync_copy(zeros_hbm.at[pl.ds(sid * rows, rows)],
                    deg_sh.at[pl.ds(sid * rows, rows)])
    pltpu.sync_copy(edge_hbm.at[1, pl.ds(wid * EPN, EPN)], dst_v)
    pltpu.sync_copy(ones_hbm, ones_v)
    plsc.subcore_barrier()

    def didx(j):
        return deg_sh.at[dst_v.at[pl.ds(j * CHUNK, CHUNK)]]

    def fire(j, carry):
        pltpu.async_copy(ones_v, didx(j), sem, add=True)
        return carry

    lax.fori_loop(0, NF, fire, 0)
    tail_dst = deg_sh.at[dst_v.at[pl.ds(NF * CHUNK, TAIL)]]
    pltpu.async_copy(ones_v.at[pl.ds(0, TAIL)], tail_dst, sem, add=True)

    def drain(j, carry):
        pltpu.make_async_copy(ones_v, didx(j), sem).wait()
        return carry

    lax.fori_loop(0, NF, drain, 0)
    pltpu.make_async_copy(ones_v.at[pl.ds(0, TAIL)], tail_dst, sem).wait()
    plsc.subcore_barrier()

    @pl.when(sid == 0)
    def _():
        pltpu.sync_copy(deg_sh.at[pl.ds(0, N)], out_hbm.at[cid])


def _enc_body(x_ref, w_ref, degp_ref, y_ref):
    deg = degp_ref[0, :] + degp_ref[1, :] + 1.0
    dinv = lax.rsqrt(deg)[:, None]
    xw = jnp.dot(x_ref[...], w_ref[...], preferred_element_type=jnp.float32)
    y_ref[...] = xw * dinv


_enc_call = pl.pallas_call(
    _enc_body,
    out_shape=jax.ShapeDtypeStruct((N, H), jnp.float32),
)


@functools.partial(
    pl.kernel,
    out_type=jax.ShapeDtypeStruct((NC, ACC_ROWS, H), jnp.float32),
    mesh=_mesh,
    compiler_params=_sc_params,
    scratch_types=[
        pltpu.VMEM((EPN,), jnp.int32),
        pltpu.VMEM((EPN,), jnp.int32),
        pltpu.VMEM((NBUF, CHUNK, H), jnp.float32),
        pltpu.VMEM((TAIL, H), jnp.float32),
        pltpu.VMEM_SHARED((ACC_ROWS, H), jnp.float32),
        [pltpu.SemaphoreType.DMA] * NBUF,
        [pltpu.SemaphoreType.DMA] * NBUF,
    ],
)
def _msg_kernel(y_hbm, edge_hbm, zeros_hbm, out_hbm,
                src_v, dst_v, buf, tbuf, acc_sh, gs, ss):
    cid = lax.axis_index("c")
    sid = lax.axis_index("s")
    wid = sid * NC + cid
    rows = ROWS_PER_TILE
    pltpu.sync_copy(zeros_hbm.at[pl.ds(sid * rows, rows)],
                    acc_sh.at[pl.ds(sid * rows, rows)])
    pltpu.sync_copy(edge_hbm.at[0, pl.ds(wid * EPN, EPN)], src_v)
    pltpu.sync_copy(edge_hbm.at[1, pl.ds(wid * EPN, EPN)], dst_v)
    plsc.subcore_barrier()

    def gather(j, b):
        pltpu.async_copy(y_hbm.at[src_v.at[pl.ds(j * CHUNK, CHUNK)]],
                         buf.at[b], gs[b])

    def gwait(j, b):
        pltpu.make_async_copy(y_hbm.at[src_v.at[pl.ds(j * CHUNK, CHUNK)]],
                              buf.at[b], gs[b]).wait()

    def scat(j, b):
        pltpu.async_copy(buf.at[b], acc_sh.at[dst_v.at[pl.ds(j * CHUNK, CHUNK)]],
                         ss[b], add=True)

    def swait(j, b):
        pltpu.make_async_copy(buf.at[b],
                              acc_sh.at[dst_v.at[pl.ds(j * CHUNK, CHUNK)]],
                              ss[b]).wait()

    for b in range(GDEPTH):
        gather(b, b)
    for b in range(GDEPTH):
        gwait(b, b)
        scat(b, b)
        gather(b + GDEPTH, b + GDEPTH)
    for b in range(GDEPTH, NBUF):
        gwait(b, b)
        scat(b, b)
        swait(b - GDEPTH, b - GDEPTH)
        gather(b + GDEPTH, b - GDEPTH)

    def body(io, carry):
        for b in range(NBUF):
            j = io * NBUF + b
            bn = (b + GDEPTH) % NBUF
            gwait(j, b)
            scat(j, b)
            swait(j - GDEPTH, bn)
            gather(j + GDEPTH, bn)
        return carry

    lax.fori_loop(1, NF // NBUF, body, 0)
    for j in range(NF - NF % NBUF, NF):
        b = j % NBUF
        gwait(j, b)
        scat(j, b)
        if j + GDEPTH < NF:
            bn = (j + GDEPTH) % NBUF
            swait(j - GDEPTH, bn)
            gather(j + GDEPTH, bn)
    for k in range(NBUF):
        j = NF - NBUF + k
        swait(j, j % NBUF)
    pltpu.sync_copy(y_hbm.at[src_v.at[pl.ds(NF * CHUNK, TAIL)]], tbuf)
    pltpu.sync_copy(tbuf, acc_sh.at[dst_v.at[pl.ds(NF * CHUNK, TAIL)]],
                    add=True)
    plsc.subcore_barrier()
    pltpu.sync_copy(acc_sh.at[pl.ds(sid * rows, rows)],
                    out_hbm.at[cid, pl.ds(sid * rows, rows)])


def _dec_body(accp_ref, y_ref, degp_ref, benc_ref, wdec_ref, bdec_ref,
              xhat_ref, z_ref):
    deg = degp_ref[0, :] + degp_ref[1, :] + 1.0
    dinv = lax.rsqrt(deg)[:, None]
    acc = accp_ref[0, :N, :] + accp_ref[1, :N, :] + y_ref[...]
    t = acc * dinv + benc_ref[...]
    z = jnp.maximum(t, 0.0)
    z_ref[...] = z
    xhat_ref[...] = (
        jnp.dot(z, wdec_ref[...], preferred_element_type=jnp.float32)
        + bdec_ref[...]
    )


_dec_call = pl.pallas_call(
    _dec_body,
    out_shape=[
        jax.ShapeDtypeStruct((N, D), jnp.float32),
        jax.ShapeDtypeStruct((N, H), jnp.float32),
    ],
)


def kernel(x, edge_index, W_enc, b_enc, W_dec, b_dec):
    ones_h = jnp.ones((CHUNK,), jnp.float32)
    zeros1 = jnp.zeros((ACC_ROWS,), jnp.float32)
    zerosh = jnp.zeros((ACC_ROWS, H), jnp.float32)

    degp = _deg_kernel(edge_index, ones_h, zeros1)
    y = _enc_call(x, W_enc, degp)
    accp = _msg_kernel(y, edge_index, zerosh)
    x_hat, z = _dec_call(accp, y, degp,
                         b_enc.reshape(1, H), W_dec, b_dec.reshape(1, D))
    return (x_hat, z)


# speedup vs baseline: 1.7519x; 1.0480x over previous
import functools

import jax
import jax.numpy as jnp
from jax import lax
from jax.experimental import pallas as pl
from jax.experimental.pallas import tpu as pltpu
from jax.experimental.pallas import tpu_sc as plsc

N = 10000
E = 320000
D = 128
H = 16

NC = 2
NS = 16
NW = NC * NS
CHUNK = 128
GDEPTH = 6
NBUF = 2 * GDEPTH
EPN = E // NW
NF = EPN // CHUNK
TAIL = EPN - NF * CHUNK
ROWS_PER_TILE = 632
ACC_ROWS = NS * ROWS_PER_TILE

_mesh = plsc.VectorSubcoreMesh(core_axis_name="c", subcore_axis_name="s")
_sc_params = pltpu.CompilerParams(use_tc_tiling_on_sc=False)


@functools.partial(
    pl.kernel,
    out_type=jax.ShapeDtypeStruct((NC, N), jnp.float32),
    mesh=_mesh,
    compiler_params=_sc_params,
    scratch_types=[
        pltpu.VMEM((EPN,), jnp.int32),
        pltpu.VMEM((CHUNK,), jnp.float32),
        pltpu.VMEM_SHARED((ACC_ROWS,), jnp.float32),
        pltpu.SemaphoreType.DMA,
    ],
)
def _deg_kernel(edge_hbm, ones_hbm, zeros_hbm, out_hbm, dst_v, ones_v, deg_sh,
                sem):
    cid = lax.axis_index("c")
    sid = lax.axis_index("s")
    wid = sid * NC + cid
    rows = ROWS_PER_TILE
    pltpu.sync_copy(zeros_hbm.at[pl.ds(sid * rows, rows)],
                    deg_sh.at[pl.ds(sid * rows, rows)])
    pltpu.sync_copy(edge_hbm.at[1, pl.ds(wid * EPN, EPN)], dst_v)
    pltpu.sync_copy(ones_hbm, ones_v)
    plsc.subcore_barrier()

    def didx(j):
        return deg_sh.at[dst_v.at[pl.ds(j * CHUNK, CHUNK)]]

    def fire(j, carry):
        pltpu.async_copy(ones_v, didx(j), sem, add=True)
        return carry

    lax.fori_loop(0, NF, fire, 0)
    tail_dst = deg_sh.at[dst_v.at[pl.ds(NF * CHUNK, TAIL)]]
    pltpu.async_copy(ones_v.at[pl.ds(0, TAIL)], tail_dst, sem, add=True)

    def drain(j, carry):
        pltpu.make_async_copy(ones_v, didx(j), sem).wait()
        return carry

    lax.fori_loop(0, NF, drain, 0)
    pltpu.make_async_copy(ones_v.at[pl.ds(0, TAIL)], tail_dst, sem).wait()
    plsc.subcore_barrier()

    @pl.when(sid == 0)
    def _():
        pltpu.sync_copy(deg_sh.at[pl.ds(0, N)], out_hbm.at[cid])


def _enc_body(x_ref, w_ref, degp_ref, y_ref):
    deg = degp_ref[0, :] + degp_ref[1, :] + 1.0
    dinv = lax.rsqrt(deg)[:, None]
    xw = jnp.dot(x_ref[...], w_ref[...], preferred_element_type=jnp.float32)
    y_ref[...] = xw * dinv


_enc_call = pl.pallas_call(
    _enc_body,
    out_shape=jax.ShapeDtypeStruct((N, H), jnp.float32),
)


@functools.partial(
    pl.kernel,
    out_type=jax.ShapeDtypeStruct((NC, ACC_ROWS, H), jnp.float32),
    mesh=_mesh,
    compiler_params=_sc_params,
    scratch_types=[
        pltpu.VMEM((EPN,), jnp.int32),
        pltpu.VMEM((EPN,), jnp.int32),
        pltpu.VMEM((NBUF, CHUNK, H), jnp.float32),
        pltpu.VMEM((TAIL, H), jnp.float32),
        pltpu.VMEM_SHARED((ACC_ROWS, H), jnp.float32),
        [pltpu.SemaphoreType.DMA] * NBUF,
        [pltpu.SemaphoreType.DMA] * NBUF,
    ],
)
def _msg_kernel(y_hbm, edge_hbm, zeros_hbm, out_hbm,
                src_v, dst_v, buf, tbuf, acc_sh, gs, ss):
    cid = lax.axis_index("c")
    sid = lax.axis_index("s")
    wid = sid * NC + cid
    rows = ROWS_PER_TILE
    pltpu.sync_copy(zeros_hbm.at[pl.ds(sid * rows, rows)],
                    acc_sh.at[pl.ds(sid * rows, rows)])
    pltpu.sync_copy(edge_hbm.at[0, pl.ds(wid * EPN, EPN)], src_v)
    pltpu.sync_copy(edge_hbm.at[1, pl.ds(wid * EPN, EPN)], dst_v)
    plsc.subcore_barrier()

    def gather(j, b):
        pltpu.async_copy(y_hbm.at[src_v.at[pl.ds(j * CHUNK, CHUNK)]],
                         buf.at[b], gs[b])

    def gwait(j, b):
        pltpu.make_async_copy(y_hbm.at[src_v.at[pl.ds(j * CHUNK, CHUNK)]],
                              buf.at[b], gs[b]).wait()

    def scat(j, b):
        pltpu.async_copy(buf.at[b], acc_sh.at[dst_v.at[pl.ds(j * CHUNK, CHUNK)]],
                         ss[b], add=True)

    def swait(j, b):
        pltpu.make_async_copy(buf.at[b],
                              acc_sh.at[dst_v.at[pl.ds(j * CHUNK, CHUNK)]],
                              ss[b]).wait()

    for b in range(GDEPTH):
        gather(b, b)
    for b in range(GDEPTH):
        gwait(b, b)
        scat(b, b)
        gather(b + GDEPTH, b + GDEPTH)
    for b in range(GDEPTH, NBUF):
        gwait(b, b)
        scat(b, b)
        swait(b - GDEPTH, b - GDEPTH)
        gather(b + GDEPTH, b - GDEPTH)

    def body(io, carry):
        for b in range(NBUF):
            j = io * NBUF + b
            bn = (b + GDEPTH) % NBUF
            gwait(j, b)
            scat(j, b)
            swait(j - GDEPTH, bn)
            gather(j + GDEPTH, bn)
        return carry

    lax.fori_loop(1, NF // NBUF, body, 0)
    for j in range(NF - NF % NBUF, NF):
        b = j % NBUF
        gwait(j, b)
        scat(j, b)
        if j + GDEPTH < NF:
            bn = (j + GDEPTH) % NBUF
            swait(j - GDEPTH, bn)
            gather(j + GDEPTH, bn)
    for k in range(NBUF):
        j = NF - NBUF + k
        swait(j, j % NBUF)
    pltpu.sync_copy(y_hbm.at[src_v.at[pl.ds(NF * CHUNK, TAIL)]], tbuf)
    pltpu.sync_copy(tbuf, acc_sh.at[dst_v.at[pl.ds(NF * CHUNK, TAIL)]],
                    add=True)
    plsc.subcore_barrier()
    pltpu.sync_copy(acc_sh.at[pl.ds(sid * rows, rows)],
                    out_hbm.at[cid, pl.ds(sid * rows, rows)])


def _dec_body(accp_ref, y_ref, degp_ref, benc_ref, wdec_ref, bdec_ref,
              xhat_ref, z_ref):
    deg = degp_ref[0, :] + degp_ref[1, :] + 1.0
    dinv = lax.rsqrt(deg)[:, None]
    acc = accp_ref[0, :N, :] + accp_ref[1, :N, :] + y_ref[...]
    t = acc * dinv + benc_ref[...]
    z = jnp.maximum(t, 0.0)
    z_ref[...] = z
    xhat_ref[...] = (
        jnp.dot(z, wdec_ref[...], preferred_element_type=jnp.float32)
        + bdec_ref[...]
    )


_dec_call = pl.pallas_call(
    _dec_body,
    out_shape=[
        jax.ShapeDtypeStruct((N, D), jnp.float32),
        jax.ShapeDtypeStruct((N, H), jnp.float32),
    ],
)


def kernel(x, edge_index, W_enc, b_enc, W_dec, b_dec):
    ones_h = jnp.ones((CHUNK,), jnp.float32)
    zeros1 = jnp.zeros((ACC_ROWS,), jnp.float32)
    zerosh = jnp.zeros((ACC_ROWS, H), jnp.float32)

    degp = _deg_kernel(edge_index, ones_h, zeros1)
    y = _enc_call(x, W_enc, degp)
    accp = _msg_kernel(y, edge_index, zerosh)
    x_hat, z = _dec_call(accp, y, degp,
                         b_enc.reshape(1, H), W_dec, b_dec.reshape(1, D))
    return (x_hat, z)
